# Initial kernel scaffold; baseline (speedup 1.0000x reference)
import jax
import jax.numpy as jnp
from jax.experimental import pallas as pl


def kernel(x, adj_t, W1, b1, g1, be1, W2, b2, g2, be2, W3, b3, g3, be3, W4, b4):
    raise NotImplementedError("write your pallas kernel here")


# speedup vs baseline: 6.1036x; 6.1036x over previous
import functools

import jax
import jax.numpy as jnp
from jax import lax
from jax.experimental import pallas as pl
from jax.experimental.pallas import tpu as pltpu
from jax.experimental.pallas import tpu_sc as plsc

N = 10000
E = 320000
D = 128
BN_EPS = 1e-5

NC = 2
NS = 16
NW = NC * NS
NP = 10240
RPT = NP // NS
RB = 64
NRC = RPT // RB
EPW = E // NW
CH = 128
NFULL = EPW // CH
TAIL = EPW - NFULL * CH


def _make_agg():
    mesh = plsc.VectorSubcoreMesh(core_axis_name="c", subcore_axis_name="s")

    @functools.partial(
        pl.kernel,
        out_type=jax.ShapeDtypeStruct((2 * NP, D), jnp.float32),
        mesh=mesh,
        scratch_types=[
            pltpu.VMEM_SHARED((NP, D), jnp.float32),
            pltpu.VMEM((CH,), jnp.int32),
            pltpu.VMEM((CH,), jnp.int32),
            pltpu.VMEM((CH, D), jnp.float32),
            pltpu.VMEM((TAIL,), jnp.int32),
            pltpu.VMEM((TAIL,), jnp.int32),
            pltpu.VMEM((TAIL, D), jnp.float32),
            pltpu.VMEM((RB, D), jnp.float32),
            pltpu.SemaphoreType.DMA,
        ],
    )
    def agg(h_hbm, z_hbm, src_hbm, dst_hbm, out_hbm,
            shared, sidx, didx, rows, sidx_t, didx_t, rows_t, rbuf, sem):
        c = lax.axis_index("c")
        s = lax.axis_index("s")
        wid = s * NC + c

        def init_body(k, carry):
            start = s * RPT + k * RB

            @pl.when(c == 0)
            def _():
                pltpu.sync_copy(h_hbm.at[pl.ds(start, RB)], rbuf)

            @pl.when(c == 1)
            def _():
                pltpu.sync_copy(z_hbm.at[pl.ds(start, RB)], rbuf)

            pltpu.sync_copy(rbuf, shared.at[pl.ds(start, RB)])
            return carry

        lax.fori_loop(0, NRC, init_body, 0)
        plsc.subcore_barrier()

        ebase = wid * EPW

        def body(i, carry):
            b = ebase + i * CH
            pltpu.sync_copy(src_hbm.at[pl.ds(b, CH)], sidx)
            pltpu.async_copy(h_hbm.at[sidx], rows, sem).wait()
            pltpu.sync_copy(dst_hbm.at[pl.ds(b, CH)], didx)
            pltpu.sync_copy(rows, shared.at[didx], add=True)
            return carry

        lax.fori_loop(0, NFULL, body, 0)

        bt = ebase + NFULL * CH
        pltpu.sync_copy(src_hbm.at[pl.ds(bt, TAIL)], sidx_t)
        pltpu.async_copy(h_hbm.at[sidx_t], rows_t, sem).wait()
        pltpu.sync_copy(dst_hbm.at[pl.ds(bt, TAIL)], didx_t)
        pltpu.sync_copy(rows_t, shared.at[didx_t], add=True)

        plsc.subcore_barrier()

        def out_body(k, carry):
            start = s * RPT + k * RB
            pltpu.sync_copy(shared.at[pl.ds(start, RB)], rbuf)
            pltpu.sync_copy(rbuf, out_hbm.at[pl.ds(c * NP + start, RB)])
            return carry

        lax.fori_loop(0, NRC, out_body, 0)

    return agg


_agg = _make_agg()


def _bn_relu(t):
    rid = lax.broadcasted_iota(jnp.int32, (NP, 1), 0)
    m = (rid < N).astype(jnp.float32)
    tm = t * m
    s1 = jnp.sum(tm, axis=0, keepdims=True)
    s2 = jnp.sum(tm * tm, axis=0, keepdims=True)
    mu = s1 / N
    var = s2 / N - mu * mu
    return m, mu, lax.rsqrt(var + BN_EPS)


def _dense_body(p_ref, w_ref, b_ref, g_ref, be_ref, o_ref):
    u = p_ref[:NP, :] + p_ref[NP:, :]
    t = jnp.dot(u, w_ref[...], preferred_element_type=jnp.float32) + b_ref[...]
    m, mu, rstd = _bn_relu(t)
    y = g_ref[...] * (t - mu) * rstd + be_ref[...]
    o_ref[...] = jnp.maximum(y, 0.0) * m


def _dense3_body(p_ref, w_ref, b_ref, g_ref, be_ref, w4_ref, b4_ref, o_ref):
    u = p_ref[:NP, :] + p_ref[NP:, :]
    t = jnp.dot(u, w_ref[...], preferred_element_type=jnp.float32) + b_ref[...]
    m, mu, rstd = _bn_relu(t)
    y = jnp.maximum(g_ref[...] * (t - mu) * rstd + be_ref[...], 0.0)
    o_ref[...] = (jnp.dot(y, w4_ref[...], preferred_element_type=jnp.float32)
                  + b4_ref[...])


def _dense(p, W, b, g, be):
    return pl.pallas_call(
        _dense_body,
        out_shape=jax.ShapeDtypeStruct((NP, D), jnp.float32),
    )(p, W, b.reshape(1, D), g.reshape(1, D), be.reshape(1, D))


def _dense3(p, W, b, g, be, W4, b4):
    return pl.pallas_call(
        _dense3_body,
        out_shape=jax.ShapeDtypeStruct((NP, D), jnp.float32),
    )(p, W, b.reshape(1, D), g.reshape(1, D), be.reshape(1, D),
      W4, b4.reshape(1, D))


def kernel(x, adj_t, W1, b1, g1, be1, W2, b2, g2, be2, W3, b3, g3, be3, W4, b4):
    src = adj_t[0]
    dst = adj_t[1]
    xp = jnp.pad(x, ((0, NP - N), (0, 0)))
    z = jnp.zeros((NP, D), jnp.float32)
    p = _agg(xp, z, src, dst)
    h = _dense(p, W1, b1, g1, be1)
    p = _agg(h, z, src, dst)
    h = _dense(p, W2, b2, g2, be2)
    p = _agg(h, z, src, dst)
    out = _dense3(p, W3, b3, g3, be3, W4, b4)
    return out[:N]


# speedup vs baseline: 9.9902x; 1.6368x over previous
import functools

import jax
import jax.numpy as jnp
from jax import lax
from jax.experimental import pallas as pl
from jax.experimental.pallas import tpu as pltpu
from jax.experimental.pallas import tpu_sc as plsc

N = 10000
E = 320000
D = 128
BN_EPS = 1e-5

NC = 2
NS = 16
NW = NC * NS
NP = 10240
RPT = NP // NS
RB = 64
NRC = RPT // RB
EPW = E // NW
CH = 128
NFULL = EPW // CH
TAIL = EPW - NFULL * CH


def _make_agg():
    mesh = plsc.VectorSubcoreMesh(core_axis_name="c", subcore_axis_name="s")

    @functools.partial(
        pl.kernel,
        out_type=jax.ShapeDtypeStruct((2 * NP, D), jnp.float32),
        mesh=mesh,
        scratch_types=[
            pltpu.VMEM_SHARED((NP, D), jnp.float32),
            pltpu.VMEM((CH,), jnp.int32),
            pltpu.VMEM((CH,), jnp.int32),
            pltpu.VMEM((CH,), jnp.int32),
            pltpu.VMEM((CH,), jnp.int32),
            pltpu.VMEM((CH, D), jnp.float32),
            pltpu.VMEM((CH, D), jnp.float32),
            pltpu.VMEM((TAIL,), jnp.int32),
            pltpu.VMEM((TAIL,), jnp.int32),
            pltpu.VMEM((TAIL, D), jnp.float32),
            pltpu.SemaphoreType.DMA,
            pltpu.SemaphoreType.DMA,
            pltpu.SemaphoreType.DMA,
        ],
    )
    def agg(h_hbm, z_hbm, src_hbm, dst_hbm, out_hbm,
            shared, sidx0, sidx1, didx0, didx1, rows0, rows1,
            sidx_t, didx_t, rows_t, sem0, sem1, sem_t):
        c = lax.axis_index("c")
        s = lax.axis_index("s")
        wid = s * NC + c
        ebase = wid * EPW
        sidx = (sidx0, sidx1)
        didx = (didx0, didx1)
        rows = (rows0, rows1)
        sem = (sem0, sem1)

        def fire(chunk, p):
            b = ebase + chunk * CH
            pltpu.sync_copy(src_hbm.at[pl.ds(b, CH)], sidx[p])
            pltpu.async_copy(h_hbm.at[sidx[p]], rows[p], sem[p])
            pltpu.sync_copy(dst_hbm.at[pl.ds(b, CH)], didx[p])

        def fire_tail():
            bt = ebase + NFULL * CH
            pltpu.sync_copy(src_hbm.at[pl.ds(bt, TAIL)], sidx_t)
            pltpu.async_copy(h_hbm.at[sidx_t], rows_t, sem_t)
            pltpu.sync_copy(dst_hbm.at[pl.ds(bt, TAIL)], didx_t)

        def finish(p):
            pltpu.make_async_copy(h_hbm.at[sidx[p]], rows[p], sem[p]).wait()
            pltpu.sync_copy(rows[p], shared.at[didx[p]], add=True)

        fire(0, 0)
        rstart = s * RPT

        @pl.when(c == 0)
        def _():
            pltpu.sync_copy(h_hbm.at[pl.ds(rstart, RPT)],
                            shared.at[pl.ds(rstart, RPT)])

        @pl.when(c == 1)
        def _():
            pltpu.sync_copy(z_hbm.at[pl.ds(rstart, RPT)],
                            shared.at[pl.ds(rstart, RPT)])

        plsc.subcore_barrier()

        def body(io, carry):
            fire(2 * io + 1, 1)
            finish(0)
            @pl.when(io < NFULL // 2 - 1)
            def _():
                fire(2 * io + 2, 0)

            @pl.when(io == NFULL // 2 - 1)
            def _():
                fire_tail()

            finish(1)
            return carry

        lax.fori_loop(0, NFULL // 2, body, 0)

        pltpu.make_async_copy(h_hbm.at[sidx_t], rows_t, sem_t).wait()
        pltpu.sync_copy(rows_t, shared.at[didx_t], add=True)

        plsc.subcore_barrier()

        pltpu.sync_copy(shared.at[pl.ds(rstart, RPT)],
                        out_hbm.at[pl.ds(c * NP + rstart, RPT)])

    return agg


_agg = _make_agg()


def _bn_relu(t):
    rid = lax.broadcasted_iota(jnp.int32, (NP, 1), 0)
    m = (rid < N).astype(jnp.float32)
    tm = t * m
    s1 = jnp.sum(tm, axis=0, keepdims=True)
    s2 = jnp.sum(tm * tm, axis=0, keepdims=True)
    mu = s1 / N
    var = s2 / N - mu * mu
    return m, mu, lax.rsqrt(var + BN_EPS)


def _dense_body(p_ref, w_ref, b_ref, g_ref, be_ref, o_ref):
    u = p_ref[:NP, :] + p_ref[NP:, :]
    t = jnp.dot(u, w_ref[...], preferred_element_type=jnp.float32) + b_ref[...]
    m, mu, rstd = _bn_relu(t)
    y = g_ref[...] * (t - mu) * rstd + be_ref[...]
    o_ref[...] = jnp.maximum(y, 0.0) * m


def _dense3_body(p_ref, w_ref, b_ref, g_ref, be_ref, w4_ref, b4_ref, o_ref):
    u = p_ref[:NP, :] + p_ref[NP:, :]
    t = jnp.dot(u, w_ref[...], preferred_element_type=jnp.float32) + b_ref[...]
    m, mu, rstd = _bn_relu(t)
    y = jnp.maximum(g_ref[...] * (t - mu) * rstd + be_ref[...], 0.0)
    o_ref[...] = (jnp.dot(y, w4_ref[...], preferred_element_type=jnp.float32)
                  + b4_ref[...])


def _dense(p, W, b, g, be):
    return pl.pallas_call(
        _dense_body,
        out_shape=jax.ShapeDtypeStruct((NP, D), jnp.float32),
    )(p, W, b.reshape(1, D), g.reshape(1, D), be.reshape(1, D))


def _dense3(p, W, b, g, be, W4, b4):
    return pl.pallas_call(
        _dense3_body,
        out_shape=jax.ShapeDtypeStruct((NP, D), jnp.float32),
    )(p, W, b.reshape(1, D), g.reshape(1, D), be.reshape(1, D),
      W4, b4.reshape(1, D))


def kernel(x, adj_t, W1, b1, g1, be1, W2, b2, g2, be2, W3, b3, g3, be3, W4, b4):
    src = adj_t[0]
    dst = adj_t[1]
    xp = jnp.pad(x, ((0, NP - N), (0, 0)))
    z = jnp.zeros((NP, D), jnp.float32)
    p = _agg(xp, z, src, dst)
    h = _dense(p, W1, b1, g1, be1)
    p = _agg(h, z, src, dst)
    h = _dense(p, W2, b2, g2, be2)
    p = _agg(h, z, src, dst)
    out = _dense3(p, W3, b3, g3, be3, W4, b4)
    return out[:N]


# speedup vs baseline: 11.8854x; 1.1897x over previous
import functools

import jax
import jax.numpy as jnp
from jax import lax
from jax.experimental import pallas as pl
from jax.experimental.pallas import tpu as pltpu
from jax.experimental.pallas import tpu_sc as plsc

N = 10000
E = 320000
D = 128
BN_EPS = 1e-5

NC = 2
NS = 16
NW = NC * NS
NP = 10240
RPT = NP // NS
RB = 64
NRC = RPT // RB
EPW = E // NW
CH = 80
NCH = EPW // CH


def _make_agg():
    mesh = plsc.VectorSubcoreMesh(core_axis_name="c", subcore_axis_name="s")

    @functools.partial(
        pl.kernel,
        out_type=jax.ShapeDtypeStruct((2 * NP, D), jnp.float32),
        mesh=mesh,
        scratch_types=[
            pltpu.VMEM_SHARED((NP, D), jnp.float32),
            pltpu.VMEM((EPW,), jnp.int32),
            pltpu.VMEM((NCH, CH), jnp.int32),
            pltpu.VMEM((CH, D), jnp.float32),
            pltpu.VMEM((CH, D), jnp.float32),
            pltpu.SemaphoreType.DMA,
            pltpu.SemaphoreType.DMA,
            pltpu.SemaphoreType.DMA,
        ],
    )
    def agg(h_hbm, z_hbm, src_hbm, dst_hbm, out_hbm,
            shared, sidx, didx, rows0, rows1, sem0, sem1, sem_i):
        c = lax.axis_index("c")
        s = lax.axis_index("s")
        wid = s * NC + c
        rows = (rows0, rows1)
        sem = (sem0, sem1)

        pltpu.async_copy(src_hbm.at[wid], sidx, sem_i)
        pltpu.async_copy(dst_hbm.at[wid], didx, sem_i)

        rstart = s * RPT

        @pl.when(c == 0)
        def _():
            pltpu.sync_copy(h_hbm.at[pl.ds(rstart, RPT)],
                            shared.at[pl.ds(rstart, RPT)])

        @pl.when(c == 1)
        def _():
            pltpu.sync_copy(z_hbm.at[pl.ds(rstart, RPT)],
                            shared.at[pl.ds(rstart, RPT)])

        pltpu.make_async_copy(src_hbm.at[wid], sidx, sem_i).wait()
        pltpu.make_async_copy(dst_hbm.at[wid], didx, sem_i).wait()

        def fire(chunk, p):
            pltpu.async_copy(h_hbm.at[sidx.at[pl.ds(chunk * CH, CH)]],
                             rows[p], sem[p])

        def finish(chunk, p):
            pltpu.make_async_copy(h_hbm.at[sidx.at[pl.ds(chunk * CH, CH)]],
                                  rows[p], sem[p]).wait()
            pltpu.sync_copy(rows[p], shared.at[didx.at[chunk]], add=True)

        fire(0, 0)
        plsc.subcore_barrier()
        fire(1, 1)

        def body(io, carry):
            c0 = 2 * io
            finish(c0, 0)

            @pl.when(c0 + 2 < NCH)
            def _():
                fire(c0 + 2, 0)

            finish(c0 + 1, 1)

            @pl.when(c0 + 3 < NCH)
            def _():
                fire(c0 + 3, 1)

            return carry

        lax.fori_loop(0, NCH // 2, body, 0)
        finish(NCH - 1, 0)

        plsc.subcore_barrier()

        pltpu.sync_copy(shared.at[pl.ds(rstart, RPT)],
                        out_hbm.at[pl.ds(c * NP + rstart, RPT)])

    return agg


_agg = _make_agg()


def _bn_relu(t):
    rid = lax.broadcasted_iota(jnp.int32, (NP, 1), 0)
    m = (rid < N).astype(jnp.float32)
    tm = t * m
    s1 = jnp.sum(tm, axis=0, keepdims=True)
    s2 = jnp.sum(tm * tm, axis=0, keepdims=True)
    mu = s1 / N
    var = s2 / N - mu * mu
    return m, mu, lax.rsqrt(var + BN_EPS)


def _dense_body(p_ref, w_ref, b_ref, g_ref, be_ref, o_ref):
    u = p_ref[:NP, :] + p_ref[NP:, :]
    t = jnp.dot(u, w_ref[...], preferred_element_type=jnp.float32) + b_ref[...]
    m, mu, rstd = _bn_relu(t)
    y = g_ref[...] * (t - mu) * rstd + be_ref[...]
    o_ref[...] = jnp.maximum(y, 0.0) * m


def _dense3_body(p_ref, w_ref, b_ref, g_ref, be_ref, w4_ref, b4_ref, o_ref):
    u = p_ref[:NP, :] + p_ref[NP:, :]
    t = jnp.dot(u, w_ref[...], preferred_element_type=jnp.float32) + b_ref[...]
    m, mu, rstd = _bn_relu(t)
    y = jnp.maximum(g_ref[...] * (t - mu) * rstd + be_ref[...], 0.0)
    o_ref[...] = (jnp.dot(y, w4_ref[...], preferred_element_type=jnp.float32)
                  + b4_ref[...])


def _dense(p, W, b, g, be):
    return pl.pallas_call(
        _dense_body,
        out_shape=jax.ShapeDtypeStruct((NP, D), jnp.float32),
    )(p, W, b.reshape(1, D), g.reshape(1, D), be.reshape(1, D))


def _dense3(p, W, b, g, be, W4, b4):
    return pl.pallas_call(
        _dense3_body,
        out_shape=jax.ShapeDtypeStruct((NP, D), jnp.float32),
    )(p, W, b.reshape(1, D), g.reshape(1, D), be.reshape(1, D),
      W4, b4.reshape(1, D))


def kernel(x, adj_t, W1, b1, g1, be1, W2, b2, g2, be2, W3, b3, g3, be3, W4, b4):
    src = adj_t[0].reshape(NW, EPW)
    dst = adj_t[1].reshape(NW, NCH, CH)
    xp = jnp.pad(x, ((0, NP - N), (0, 0)))
    z = jnp.zeros((NP, D), jnp.float32)
    p = _agg(xp, z, src, dst)
    h = _dense(p, W1, b1, g1, be1)
    p = _agg(h, z, src, dst)
    h = _dense(p, W2, b2, g2, be2)
    p = _agg(h, z, src, dst)
    out = _dense3(p, W3, b3, g3, be3, W4, b4)
    return out[:N]
